# baseline (device time: 59919 ns/iter reference)
import jax
import jax.numpy as jnp
from jax import lax
from jax.experimental import pallas as pl
from jax.experimental.pallas import tpu as pltpu

N_DEV = 8
P = 2
R, L = 0, 1


def kernel(x, W1, W2):
    m, k = x.shape
    _, d = W1.shape
    _, f = W2.shape
    chunk = m // N_DEV
    hd = d // 2
    rh = chunk // P

    def body(x_ref, w1_ref, w2_ref, out_ref, part_ref,
             rs_buf, ag_buf, rs_s, rs_r, ag_s, ag_r):
        i = lax.axis_index("i")
        left = lax.rem(i - 1 + N_DEV, N_DEV)
        right = lax.rem(i + 1, N_DEV)

        barrier_sem = pltpu.get_barrier_semaphore()
        for nbr in (left, right):
            pl.semaphore_signal(
                barrier_sem, inc=1,
                device_id=(nbr,), device_id_type=pl.DeviceIdType.MESH,
            )
        pl.semaphore_wait(barrier_sem, 2)

        w1 = w1_ref[...].astype(jnp.bfloat16)
        w2R = w2_ref[:hd, :].astype(jnp.bfloat16)
        w2L = w2_ref[hd:, :].astype(jnp.bfloat16)

        def pchunk(c):
            xa = x_ref[pl.ds(c * chunk, chunk), :].astype(jnp.bfloat16)
            return jnp.dot(xa, w1, preferred_element_type=jnp.float32)

        def cols(dirn):
            return slice(0, hd) if dirn == R else slice(hd, d)

        def rdma(buf, sems_s, sems_r, dirn, p, src_slot, dst_slot, step):
            rows = pl.ds(p * rh, rh)
            return pltpu.make_async_remote_copy(
                src_ref=buf.at[dirn, src_slot, rows],
                dst_ref=buf.at[dirn, dst_slot, rows],
                send_sem=sems_s.at[dirn, p, step],
                recv_sem=sems_r.at[dirn, p, step],
                device_id=(right if dirn == R else left,),
                device_id_type=pl.DeviceIdType.MESH,
            )

        sent = []

        p7 = pchunk(lax.rem(i + 7, N_DEV))
        part_ref[7] = p7
        rs_buf[R, 7] = p7[:, cols(R)].astype(jnp.bfloat16)
        for p in range(P):
            r = rdma(rs_buf, rs_s, rs_r, R, p, 7, 0, 0)
            r.start()
            sent.append(r)
        p1 = pchunk(lax.rem(i + 1, N_DEV))
        part_ref[1] = p1
        rs_buf[L, 7] = p1[:, cols(L)].astype(jnp.bfloat16)
        for p in range(P):
            r = rdma(rs_buf, rs_s, rs_r, L, p, 7, 0, 0)
            r.start()
            sent.append(r)

        for r_off in (6, 2, 5, 3, 4, 0):
            part_ref[r_off] = pchunk(lax.rem(i + r_off, N_DEV))

        accs = {}
        for s in range(N_DEV - 1):
            final = s == N_DEV - 2
            for p in range(P):
                for dirn in (R, L):
                    rdma(rs_buf, rs_s, rs_r, dirn, p, s, s, s).wait_recv()
                    r_off = (6 - s) if dirn == R else (2 + s) % N_DEV
                    rows = pl.ds(p * rh, rh)
                    acc = (rs_buf[dirn, s, rows].astype(jnp.float32)
                           + part_ref[r_off][p * rh:(p + 1) * rh, cols(dirn)])
                    if not final:
                        rs_buf[dirn, s, rows] = acc.astype(jnp.bfloat16)
                        r = rdma(rs_buf, rs_s, rs_r, dirn, p, s, s + 1, s + 1)
                        r.start()
                        sent.append(r)
                    else:
                        ag_buf[dirn, 7, rows] = acc.astype(jnp.bfloat16)
                        r = rdma(ag_buf, ag_s, ag_r, dirn, p, 7, 0, 0)
                        r.start()
                        sent.append(r)
                        accs[(dirn, p)] = acc
                if final:
                    out_ref[pl.ds(i * chunk + p * rh, rh), :] = (
                        jnp.dot(accs[(R, p)].astype(jnp.bfloat16), w2R,
                                preferred_element_type=jnp.float32)
                        + jnp.dot(accs[(L, p)].astype(jnp.bfloat16), w2L,
                                  preferred_element_type=jnp.float32)
                    )

        for s in range(N_DEV - 1):
            cR = lax.rem(i - 1 - s + 2 * N_DEV, N_DEV)
            cL = lax.rem(i + 1 + s, N_DEV)
            for p in range(P):
                for dirn in (R, L):
                    rdma(ag_buf, ag_s, ag_r, dirn, p, s, s, s).wait_recv()
                    if s < N_DEV - 2:
                        r = rdma(ag_buf, ag_s, ag_r, dirn, p, s, s + 1, s + 1)
                        r.start()
                        sent.append(r)
                rows = pl.ds(p * rh, rh)
                pieceR = jnp.dot(ag_buf[R, s, rows], w2R,
                                 preferred_element_type=jnp.float32)
                pieceL = jnp.dot(ag_buf[L, s, rows], w2L,
                                 preferred_element_type=jnp.float32)
                dR = pl.ds(cR * chunk + p * rh, rh)
                dL = pl.ds(cL * chunk + p * rh, rh)
                if s < 3:
                    out_ref[dR, :] = pieceR
                    out_ref[dL, :] = pieceL
                elif s == 3:
                    out_ref[dR, :] = pieceR + pieceL
                else:
                    out_ref[dR, :] = out_ref[dR, :] + pieceR
                    out_ref[dL, :] = out_ref[dL, :] + pieceL

        for r in sent:
            r.wait_send()

    return pl.pallas_call(
        body,
        out_shape=jax.ShapeDtypeStruct((m, f), jnp.float32),
        in_specs=[
            pl.BlockSpec(memory_space=pltpu.VMEM),
            pl.BlockSpec(memory_space=pltpu.VMEM),
            pl.BlockSpec(memory_space=pltpu.VMEM),
        ],
        out_specs=pl.BlockSpec(memory_space=pltpu.VMEM),
        scratch_shapes=[
            pltpu.VMEM((N_DEV, chunk, d), jnp.float32),
            pltpu.VMEM((2, N_DEV, chunk, hd), jnp.bfloat16),
            pltpu.VMEM((2, N_DEV, chunk, hd), jnp.bfloat16),
            pltpu.SemaphoreType.DMA((2, P, N_DEV - 1)),
            pltpu.SemaphoreType.DMA((2, P, N_DEV - 1)),
            pltpu.SemaphoreType.DMA((2, P, N_DEV - 1)),
            pltpu.SemaphoreType.DMA((2, P, N_DEV - 1)),
        ],
        compiler_params=pltpu.CompilerParams(collective_id=0),
    )(x, W1, W2)


# device time: 56528 ns/iter; 1.0600x vs baseline; 1.0600x over previous
import jax
import jax.numpy as jnp
from jax import lax
from jax.experimental import pallas as pl
from jax.experimental.pallas import tpu as pltpu

N_DEV = 8
P = 4
R, L = 0, 1


def kernel(x, W1, W2):
    m, k = x.shape
    _, d = W1.shape
    _, f = W2.shape
    chunk = m // N_DEV
    hd = d // 2
    rh = chunk // P

    def body(x_ref, w1_ref, w2_ref, out_ref, part_ref,
             rs_buf, ag_buf, rs_s, rs_r, ag_s, ag_r):
        i = lax.axis_index("i")
        left = lax.rem(i - 1 + N_DEV, N_DEV)
        right = lax.rem(i + 1, N_DEV)

        barrier_sem = pltpu.get_barrier_semaphore()
        for nbr in (left, right):
            pl.semaphore_signal(
                barrier_sem, inc=1,
                device_id=(nbr,), device_id_type=pl.DeviceIdType.MESH,
            )
        pl.semaphore_wait(barrier_sem, 2)

        w1 = w1_ref[...].astype(jnp.bfloat16)
        w1R = w1[:, :hd]
        w1L = w1[:, hd:]
        w2R = w2_ref[:hd, :].astype(jnp.bfloat16)
        w2L = w2_ref[hd:, :].astype(jnp.bfloat16)

        def xchunk(c):
            return x_ref[pl.ds(c * chunk, chunk), :].astype(jnp.bfloat16)

        def cols(dirn):
            return slice(0, hd) if dirn == R else slice(hd, d)

        def rdma(buf, sems_s, sems_r, dirn, p, src_slot, dst_slot, step):
            rows = pl.ds(p * rh, rh)
            return pltpu.make_async_remote_copy(
                src_ref=buf.at[dirn, src_slot, rows],
                dst_ref=buf.at[dirn, dst_slot, rows],
                send_sem=sems_s.at[dirn, p, step],
                recv_sem=sems_r.at[dirn, p, step],
                device_id=(right if dirn == R else left,),
                device_id_type=pl.DeviceIdType.MESH,
            )

        sent = []

        x7 = xchunk(lax.rem(i + 7, N_DEV))
        rs_buf[R, 7] = jnp.dot(x7, w1R,
                               preferred_element_type=jnp.float32
                               ).astype(jnp.bfloat16)
        for p in range(P):
            r = rdma(rs_buf, rs_s, rs_r, R, p, 7, 0, 0)
            r.start()
            sent.append(r)
        x1 = xchunk(lax.rem(i + 1, N_DEV))
        rs_buf[L, 7] = jnp.dot(x1, w1L,
                               preferred_element_type=jnp.float32
                               ).astype(jnp.bfloat16)
        for p in range(P):
            r = rdma(rs_buf, rs_s, rs_r, L, p, 7, 0, 0)
            r.start()
            sent.append(r)

        part_ref[7, :, hd:] = jnp.dot(
            x7, w1L, preferred_element_type=jnp.float32).astype(jnp.bfloat16)
        part_ref[1, :, :hd] = jnp.dot(
            x1, w1R, preferred_element_type=jnp.float32).astype(jnp.bfloat16)
        for r_off in (6, 2, 5, 3, 4, 0):
            part_ref[r_off] = jnp.dot(
                xchunk(lax.rem(i + r_off, N_DEV)), w1,
                preferred_element_type=jnp.float32).astype(jnp.bfloat16)

        for s in range(N_DEV - 1):
            final = s == N_DEV - 2
            for p in range(P):
                rows = pl.ds(p * rh, rh)
                for dirn in (R, L):
                    rdma(rs_buf, rs_s, rs_r, dirn, p, s, s, s).wait_recv()
                    r_off = (6 - s) if dirn == R else (2 + s) % N_DEV
                    acc = (rs_buf[dirn, s, rows]
                           + part_ref[r_off][p * rh:(p + 1) * rh, cols(dirn)])
                    if not final:
                        rs_buf[dirn, s, rows] = acc
                        r = rdma(rs_buf, rs_s, rs_r, dirn, p, s, s + 1, s + 1)
                    else:
                        ag_buf[dirn, 7, rows] = acc
                        r = rdma(ag_buf, ag_s, ag_r, dirn, p, 7, 0, 0)
                    r.start()
                    sent.append(r)
        out_ref[pl.ds(i * chunk, chunk), :] = (
            jnp.dot(ag_buf[R, 7], w2R, preferred_element_type=jnp.float32)
            + jnp.dot(ag_buf[L, 7], w2L, preferred_element_type=jnp.float32)
        )

        for s in range(N_DEV - 1):
            for p in range(P):
                for dirn in (R, L):
                    rdma(ag_buf, ag_s, ag_r, dirn, p, s, s, s).wait_recv()
                    if s < N_DEV - 2:
                        r = rdma(ag_buf, ag_s, ag_r, dirn, p, s, s + 1, s + 1)
                        r.start()
                        sent.append(r)
            pieceR = jnp.dot(ag_buf[R, s], w2R,
                             preferred_element_type=jnp.float32)
            pieceL = jnp.dot(ag_buf[L, s], w2L,
                             preferred_element_type=jnp.float32)
            cR = lax.rem(i - 1 - s + 2 * N_DEV, N_DEV)
            cL = lax.rem(i + 1 + s, N_DEV)
            dR = pl.ds(cR * chunk, chunk)
            dL = pl.ds(cL * chunk, chunk)
            if s < 3:
                out_ref[dR, :] = pieceR
                out_ref[dL, :] = pieceL
            elif s == 3:
                out_ref[dR, :] = pieceR + pieceL
            else:
                out_ref[dR, :] = out_ref[dR, :] + pieceR
                out_ref[dL, :] = out_ref[dL, :] + pieceL

        for r in sent:
            r.wait_send()

    return pl.pallas_call(
        body,
        out_shape=jax.ShapeDtypeStruct((m, f), jnp.float32),
        in_specs=[
            pl.BlockSpec(memory_space=pltpu.VMEM),
            pl.BlockSpec(memory_space=pltpu.VMEM),
            pl.BlockSpec(memory_space=pltpu.VMEM),
        ],
        out_specs=pl.BlockSpec(memory_space=pltpu.VMEM),
        scratch_shapes=[
            pltpu.VMEM((N_DEV, chunk, d), jnp.bfloat16),
            pltpu.VMEM((2, N_DEV, chunk, hd), jnp.bfloat16),
            pltpu.VMEM((2, N_DEV, chunk, hd), jnp.bfloat16),
            pltpu.SemaphoreType.DMA((2, P, N_DEV - 1)),
            pltpu.SemaphoreType.DMA((2, P, N_DEV - 1)),
            pltpu.SemaphoreType.DMA((2, P, N_DEV - 1)),
            pltpu.SemaphoreType.DMA((2, P, N_DEV - 1)),
        ],
        compiler_params=pltpu.CompilerParams(collective_id=0),
    )(x, W1, W2)


# device time: 56130 ns/iter; 1.0675x vs baseline; 1.0071x over previous
import jax
import jax.numpy as jnp
from jax import lax
from jax.experimental import pallas as pl
from jax.experimental.pallas import tpu as pltpu

N_DEV = 8
P = 4


def kernel(x, W1, W2):
    m, k = x.shape
    _, d = W1.shape
    _, f = W2.shape
    chunk = m // N_DEV
    hd = d // 2
    rh = chunk // P

    def body(x_ref, w1_ref, w2_ref, out_ref, part_ref,
             rsRA, rsLA, rsRB, rsLB, own_ref, agR, agL,
             rsRA_s, rsRA_r, rsLA_s, rsLA_r, rsRB_s, rsRB_r,
             rsLB_s, rsLB_r, agR_s, agR_r, agL_s, agL_r):
        i = lax.axis_index("i")
        left = lax.rem(i - 1 + N_DEV, N_DEV)
        right = lax.rem(i + 1, N_DEV)

        barrier_sem = pltpu.get_barrier_semaphore()
        for nbr in (left, right):
            pl.semaphore_signal(
                barrier_sem, inc=1,
                device_id=(nbr,), device_id_type=pl.DeviceIdType.MESH,
            )
        pl.semaphore_wait(barrier_sem, 2)

        w1 = w1_ref[...].astype(jnp.bfloat16)
        w1R = w1[:, :hd]
        w1L = w1[:, hd:]
        w2R = w2_ref[:hd, :].astype(jnp.bfloat16)
        w2L = w2_ref[hd:, :].astype(jnp.bfloat16)

        def xchunk(o):
            c = lax.rem(i + o, N_DEV)
            return x_ref[pl.ds(c * chunk, chunk), :].astype(jnp.bfloat16)

        def rows(p):
            return pl.ds(p * rh, rh)

        sent = []

        def piece_rdma(src, dst, ssem, rsem, dev):
            r = pltpu.make_async_remote_copy(
                src_ref=src, dst_ref=dst, send_sem=ssem, recv_sem=rsem,
                device_id=(dev,), device_id_type=pl.DeviceIdType.MESH,
            )
            return r

        def seed(buf, sems_s, sems_r, slot, dev):
            for p in range(P):
                r = piece_rdma(buf.at[slot, rows(p)], buf.at[0, rows(p)],
                               sems_s.at[p, 0], sems_r.at[p, 0], dev)
                r.start()
                sent.append(r)

        rsRA[4] = jnp.dot(xchunk(4), w1R,
                          preferred_element_type=jnp.float32
                          ).astype(jnp.bfloat16)
        seed(rsRA, rsRA_s, rsRA_r, 4, right)
        rsLA[3] = jnp.dot(xchunk(5), w1R,
                          preferred_element_type=jnp.float32
                          ).astype(jnp.bfloat16)
        seed(rsLA, rsLA_s, rsLA_r, 3, left)
        rsRB[3] = jnp.dot(xchunk(3), w1L,
                          preferred_element_type=jnp.float32
                          ).astype(jnp.bfloat16)
        seed(rsRB, rsRB_s, rsRB_r, 3, right)
        rsLB[4] = jnp.dot(xchunk(4), w1L,
                          preferred_element_type=jnp.float32
                          ).astype(jnp.bfloat16)
        seed(rsLB, rsLB_s, rsLB_r, 4, left)

        for o in (3, 6, 2, 5, 1, 7, 0):
            part_ref[o] = jnp.dot(xchunk(o), w1,
                                  preferred_element_type=jnp.float32
                                  ).astype(jnp.bfloat16)

        def colsA(a):
            return a[:, :hd]

        def colsB(a):
            return a[:, hd:]

        def rs_step(buf, ss, sr, s, p, o_add, half, dev, fwd):
            piece_rdma(buf.at[s, rows(p)], buf.at[s, rows(p)],
                       ss.at[p, s], sr.at[p, s], dev).wait_recv()
            padd = part_ref[o_add][p * rh:(p + 1) * rh, :]
            padd = colsA(padd) if half == "A" else colsB(padd)
            buf[s, rows(p)] = buf[s, rows(p)] + padd
            if fwd:
                r = piece_rdma(buf.at[s, rows(p)], buf.at[s + 1, rows(p)],
                               ss.at[p, s + 1], sr.at[p, s + 1], dev)
                r.start()
                sent.append(r)

        for s in (0, 1):
            for p in range(P):
                rs_step(rsRA, rsRA_s, rsRA_r, s, p, 3 - s, "A", right, True)
                rs_step(rsLA, rsLA_s, rsLA_r, s, p, 6 + s, "A", left, True)
                rs_step(rsRB, rsRB_s, rsRB_r, s, p, 2 - s, "B", right, True)
                rs_step(rsLB, rsLB_s, rsLB_r, s, p, 5 + s, "B", left, True)
        for p in range(P):
            rs_step(rsRA, rsRA_s, rsRA_r, 2, p, 1, "A", right, True)
            rs_step(rsLB, rsLB_s, rsLB_r, 2, p, 7, "B", left, True)

        for p in range(P):
            rp = rows(p)
            piece_rdma(rsRA.at[3, rp], rsRA.at[3, rp],
                       rsRA_s.at[p, 3], rsRA_r.at[p, 3], right).wait_recv()
            piece_rdma(rsLA.at[2, rp], rsLA.at[2, rp],
                       rsLA_s.at[p, 2], rsLA_r.at[p, 2], left).wait_recv()
            own_ref[rp, :hd] = (rsRA[3, rp] + rsLA[2, rp]
                                + part_ref[0][p * rh:(p + 1) * rh, :hd])
            piece_rdma(rsLB.at[3, rp], rsLB.at[3, rp],
                       rsLB_s.at[p, 3], rsLB_r.at[p, 3], left).wait_recv()
            piece_rdma(rsRB.at[2, rp], rsRB.at[2, rp],
                       rsRB_s.at[p, 2], rsRB_r.at[p, 2], right).wait_recv()
            own_ref[rp, hd:] = (rsLB[3, rp] + rsRB[2, rp]
                                + part_ref[0][p * rh:(p + 1) * rh, hd:])
            for dev, sems_s, sems_r, buf in ((right, agR_s, agR_r, agR),
                                             (left, agL_s, agL_r, agL)):
                r = piece_rdma(own_ref.at[rp], buf.at[0, rows(p)],
                               sems_s.at[p, 0], sems_r.at[p, 0], dev)
                r.start()
                sent.append(r)

        out_ref[pl.ds(i * chunk, chunk), :] = (
            jnp.dot(own_ref[:, :hd], w2R, preferred_element_type=jnp.float32)
            + jnp.dot(own_ref[:, hd:], w2L, preferred_element_type=jnp.float32)
        )

        for s in range(4):
            for p in range(P):
                for buf, ss, sr, dev, half0 in (
                        (agR, agR_s, agR_r, right, 0),
                        (agL, agL_s, agL_r, left, hd)):
                    if s == 3:
                        cs = pl.ds(half0, hd)
                        piece_rdma(buf.at[s, rows(p), cs],
                                   buf.at[s, rows(p), cs],
                                   ss.at[p, s], sr.at[p, s], dev).wait_recv()
                    else:
                        piece_rdma(buf.at[s, rows(p)], buf.at[s, rows(p)],
                                   ss.at[p, s], sr.at[p, s], dev).wait_recv()
                    if s < 2:
                        r = piece_rdma(buf.at[s, rows(p)],
                                       buf.at[s + 1, rows(p)],
                                       ss.at[p, s + 1], sr.at[p, s + 1], dev)
                        r.start()
                        sent.append(r)
                    elif s == 2:
                        cs = pl.ds(half0, hd)
                        r = piece_rdma(buf.at[s, rows(p), cs],
                                       buf.at[s + 1, rows(p), cs],
                                       ss.at[p, s + 1], sr.at[p, s + 1], dev)
                        r.start()
                        sent.append(r)
            if s < 3:
                cR = lax.rem(i - 1 - s + N_DEV, N_DEV)
                cL = lax.rem(i + 1 + s, N_DEV)
                out_ref[pl.ds(cR * chunk, chunk), :] = (
                    jnp.dot(agR[s, :, :hd], w2R,
                            preferred_element_type=jnp.float32)
                    + jnp.dot(agR[s, :, hd:], w2L,
                              preferred_element_type=jnp.float32)
                )
                out_ref[pl.ds(cL * chunk, chunk), :] = (
                    jnp.dot(agL[s, :, :hd], w2R,
                            preferred_element_type=jnp.float32)
                    + jnp.dot(agL[s, :, hd:], w2L,
                              preferred_element_type=jnp.float32)
                )
            else:
                c4 = lax.rem(i + 4, N_DEV)
                out_ref[pl.ds(c4 * chunk, chunk), :] = (
                    jnp.dot(agR[3, :, :hd], w2R,
                            preferred_element_type=jnp.float32)
                    + jnp.dot(agL[3, :, hd:], w2L,
                              preferred_element_type=jnp.float32)
                )

        for r in sent:
            r.wait_send()

    return pl.pallas_call(
        body,
        out_shape=jax.ShapeDtypeStruct((m, f), jnp.float32),
        in_specs=[
            pl.BlockSpec(memory_space=pltpu.VMEM),
            pl.BlockSpec(memory_space=pltpu.VMEM),
            pl.BlockSpec(memory_space=pltpu.VMEM),
        ],
        out_specs=pl.BlockSpec(memory_space=pltpu.VMEM),
        scratch_shapes=[
            pltpu.VMEM((N_DEV, chunk, d), jnp.bfloat16),
            pltpu.VMEM((5, chunk, hd), jnp.bfloat16),
            pltpu.VMEM((4, chunk, hd), jnp.bfloat16),
            pltpu.VMEM((4, chunk, hd), jnp.bfloat16),
            pltpu.VMEM((5, chunk, hd), jnp.bfloat16),
            pltpu.VMEM((chunk, d), jnp.bfloat16),
            pltpu.VMEM((4, chunk, d), jnp.bfloat16),
            pltpu.VMEM((4, chunk, d), jnp.bfloat16),
            pltpu.SemaphoreType.DMA((P, 4)),
            pltpu.SemaphoreType.DMA((P, 4)),
            pltpu.SemaphoreType.DMA((P, 3)),
            pltpu.SemaphoreType.DMA((P, 3)),
            pltpu.SemaphoreType.DMA((P, 3)),
            pltpu.SemaphoreType.DMA((P, 3)),
            pltpu.SemaphoreType.DMA((P, 4)),
            pltpu.SemaphoreType.DMA((P, 4)),
            pltpu.SemaphoreType.DMA((P, 4)),
            pltpu.SemaphoreType.DMA((P, 4)),
            pltpu.SemaphoreType.DMA((P, 4)),
            pltpu.SemaphoreType.DMA((P, 4)),
        ],
        compiler_params=pltpu.CompilerParams(collective_id=0),
    )(x, W1, W2)
